# baseline (speedup 1.0000x reference)
import functools

import jax
import jax.numpy as jnp
from jax import lax
from jax.experimental import pallas as pl
from jax.experimental.pallas import tpu as pltpu
from jax.experimental.pallas import tpu_sc as plsc

N_ATOMS = 500000
N_TORSIONS = 2000000
NT_HALF = N_TORSIONS // 4
L = 16
C = 800
NCHUNKS = NT_HALF // C
NC = 2
NS = 16
NW = NC * NS
ITERS_PER_W = -(-NCHUNKS // NW)
NPIECE = 2400
NPIECES = (3 * N_ATOMS) // NPIECE

_PI = 3.14159265358979


def _rsqrt(y):
    i = plsc.bitcast(y, jnp.int32)
    i = jnp.int32(0x5F3759DF) - (i >> 1)
    r = plsc.bitcast(i, jnp.float32)
    for _ in range(3):
        r = r * (1.5 - 0.5 * y * r * r)
    return r


def _acos(x):
    t = jnp.abs(x)
    y = 1.0 - t
    s = y * _rsqrt(jnp.maximum(y, 1e-30))
    p = -0.0012624911
    for a in (0.0066700901, -0.0170881256, 0.0308918810, -0.0501743046,
              0.0889789874, -0.2145988016, 1.5707963050):
        p = p * t + a
    r = s * p
    return jnp.where(x >= 0, r, _PI - r)


def _torsion_sc_kernel(tors_hbm, coords_hbm, out_hbm,
                       table_sh, idx_v, gidx0, gidx1, gidx2, gidx3,
                       rows0, rows1, rows2, rows3,
                       out_v, sem0, sem1, sem2, sem3):
    gidx = [gidx0, gidx1, gidx2, gidx3]
    rows = [rows0, rows1, rows2, rows3]
    sems = [sem0, sem1, sem2, sem3]
    rows_v = rows0
    cid = lax.axis_index("c")
    sid = lax.axis_index("s")
    wid = sid * NC + cid

    def stage(q0, carry):
        q = sid + q0 * NS

        @pl.when(q < NPIECES)
        def _():
            pltpu.sync_copy(coords_hbm.at[pl.ds(q * NPIECE, NPIECE)],
                            rows_v.at[pl.ds(0, NPIECE)])
            pltpu.sync_copy(rows_v.at[pl.ds(0, NPIECE)],
                            table_sh.at[pl.ds(q * NPIECE, NPIECE)])

        return carry

    lax.fori_loop(0, -(-NPIECES // NS), stage, 0)
    plsc.subcore_barrier()

    def do_chunk(g):
        for p in range(4):
            pltpu.sync_copy(tors_hbm.at[pl.ds(p * NT_HALF + g * C, C)],
                            idx_v.at[pl.ds(p * C, C)])

        def build(b, carry):
            o = b * L
            for p in range(4):
                ap = idx_v[pl.ds(p * C + o, L)]
                for c in range(3):
                    gidx[p][pl.ds(c * C + o, L)] = ap + c * N_ATOMS
            return carry

        lax.fori_loop(0, C // L, build, 0)

        hs = [pltpu.async_copy(table_sh.at[gidx[p]], rows[p], sems[p])
              for p in range(4)]
        for h in hs:
            h.wait()

        def body(b, carry):
            o = b * L
            r = [[rows[p][pl.ds(c * C + o, L)]
                  for c in range(3)] for p in range(4)]
            b1 = [r[1][c] - r[0][c] for c in range(3)]
            b2 = [r[2][c] - r[1][c] for c in range(3)]
            b3 = [r[3][c] - r[2][c] for c in range(3)]
            n1 = [b1[1] * b2[2] - b1[2] * b2[1],
                  b1[2] * b2[0] - b1[0] * b2[2],
                  b1[0] * b2[1] - b1[1] * b2[0]]
            n2 = [b2[1] * b3[2] - b2[2] * b3[1],
                  b2[2] * b3[0] - b2[0] * b3[2],
                  b2[0] * b3[1] - b2[1] * b3[0]]
            d = n1[0] * n2[0] + n1[1] * n2[1] + n1[2] * n2[2]
            n1sq = n1[0] * n1[0] + n1[1] * n1[1] + n1[2] * n1[2]
            n2sq = n2[0] * n2[0] + n2[1] * n2[1] + n2[2] * n2[2]
            sdot = n1[0] * b3[0] + n1[1] * b3[1] + n1[2] * b3[2]
            denom = n1sq * n2sq
            sq = denom * _rsqrt(jnp.maximum(denom, 1e-35))
            cos_raw = d / sq
            cos_cl = jnp.minimum(jnp.maximum(cos_raw, -0.999999999), 0.99999999)
            is_nan = cos_raw != cos_raw
            cos = jnp.where(is_nan, cos_raw, cos_cl)
            phi = _acos(cos)
            phi = jnp.where(is_nan, cos, phi)
            phi = jnp.where(sdot > 0, phi, -phi)
            out_v[pl.ds(o, L)] = phi
            return carry

        lax.fori_loop(0, C // L, body, 0)
        pltpu.sync_copy(out_v, out_hbm.at[pl.ds(g * C, C)])

    def chunk_loop(t, carry):
        g = wid + t * NW

        @pl.when(g < NCHUNKS)
        def _():
            do_chunk(g)

        return carry

    lax.fori_loop(0, ITERS_PER_W, chunk_loop, 0)


def kernel(coords, torsions):
    coords_t = coords.T.reshape(-1)
    tors_q = [torsions[q * NT_HALF:(q + 1) * NT_HALF].T.reshape(-1)
              for q in range(4)]

    mesh = plsc.VectorSubcoreMesh(core_axis_name="c", subcore_axis_name="s")
    run = functools.partial(
        pl.kernel,
        mesh=mesh,
        compiler_params=pltpu.CompilerParams(needs_layout_passes=False,
                                             use_tc_tiling_on_sc=False),
        out_type=jax.ShapeDtypeStruct((NT_HALF,), jnp.float32),
        scratch_types=[
            pltpu.VMEM_SHARED((3 * N_ATOMS,), jnp.float32),
            pltpu.VMEM((4 * C,), jnp.int32),
            pltpu.VMEM((3 * C,), jnp.int32),
            pltpu.VMEM((3 * C,), jnp.int32),
            pltpu.VMEM((3 * C,), jnp.int32),
            pltpu.VMEM((3 * C,), jnp.int32),
            pltpu.VMEM((3 * C,), jnp.float32),
            pltpu.VMEM((3 * C,), jnp.float32),
            pltpu.VMEM((3 * C,), jnp.float32),
            pltpu.VMEM((3 * C,), jnp.float32),
            pltpu.VMEM((C,), jnp.float32),
            pltpu.SemaphoreType.DMA,
            pltpu.SemaphoreType.DMA,
            pltpu.SemaphoreType.DMA,
            pltpu.SemaphoreType.DMA,
        ],
    )(_torsion_sc_kernel)
    outs = [run(t, coords_t) for t in tors_q]
    return jnp.concatenate(outs)


# speedup vs baseline: 1.1908x; 1.1908x over previous
import functools

import jax
import jax.numpy as jnp
from jax import lax
from jax.experimental import pallas as pl
from jax.experimental.pallas import tpu as pltpu
from jax.experimental.pallas import tpu_sc as plsc

N_ATOMS = 500000
N_TORSIONS = 2000000
NT_HALF = N_TORSIONS // 2
L = 16
C = 400
NCHUNKS = NT_HALF // C
NC = 2
NS = 16
NW = NC * NS
ITERS_PER_W = -(-NCHUNKS // NW)
NPAIRS = -(-ITERS_PER_W // 2)
NPIECE = 1200
NPIECES = (3 * N_ATOMS) // NPIECE

_PI = 3.14159265358979


def _rsqrt(y):
    i = plsc.bitcast(y, jnp.int32)
    i = jnp.int32(0x5F3759DF) - (i >> 1)
    r = plsc.bitcast(i, jnp.float32)
    for _ in range(3):
        r = r * (1.5 - 0.5 * y * r * r)
    return r


def _acos(x):
    t = jnp.abs(x)
    y = 1.0 - t
    s = y * _rsqrt(jnp.maximum(y, 1e-30))
    p = -0.0012624911
    for a in (0.0066700901, -0.0170881256, 0.0308918810, -0.0501743046,
              0.0889789874, -0.2145988016, 1.5707963050):
        p = p * t + a
    r = s * p
    return jnp.where(x >= 0, r, _PI - r)


def _torsion_sc_kernel(tors_hbm, coords_hbm, out_hbm,
                       table_sh, idx_v,
                       ga0, ga1, ga2, ga3, gb0, gb1, gb2, gb3,
                       ra0, ra1, ra2, ra3, rb0, rb1, rb2, rb3,
                       out_v,
                       sa0, sa1, sa2, sa3, sb0, sb1, sb2, sb3):
    gidx = [[ga0, ga1, ga2, ga3], [gb0, gb1, gb2, gb3]]
    rows = [[ra0, ra1, ra2, ra3], [rb0, rb1, rb2, rb3]]
    sems = [[sa0, sa1, sa2, sa3], [sb0, sb1, sb2, sb3]]
    cid = lax.axis_index("c")
    sid = lax.axis_index("s")
    wid = sid * NC + cid

    def stage(q0, carry):
        q = sid + q0 * NS

        @pl.when(q < NPIECES)
        def _():
            pltpu.sync_copy(coords_hbm.at[pl.ds(q * NPIECE, NPIECE)],
                            ra0.at[pl.ds(0, NPIECE)])
            pltpu.sync_copy(ra0.at[pl.ds(0, NPIECE)],
                            table_sh.at[pl.ds(q * NPIECE, NPIECE)])

        return carry

    lax.fori_loop(0, -(-NPIECES // NS), stage, 0)
    plsc.subcore_barrier()

    def load_and_build(g, s):
        for p in range(4):
            pltpu.sync_copy(tors_hbm.at[pl.ds(p * NT_HALF + g * C, C)],
                            idx_v.at[pl.ds(p * C, C)])

        def build(b, carry):
            o = b * L
            for p in range(4):
                ap = idx_v[pl.ds(p * C + o, L)]
                for c in range(3):
                    gidx[s][p][pl.ds(c * C + o, L)] = ap + c * N_ATOMS
            return carry

        lax.fori_loop(0, C // L, build, 0)

    def fire(s):
        for p in range(4):
            pltpu.async_copy(table_sh.at[gidx[s][p]], rows[s][p], sems[s][p])

    def drain(s):
        for p in range(4):
            pltpu.make_async_copy(table_sh.at[gidx[s][p]], rows[s][p],
                                  sems[s][p]).wait()

    def compute(g, s):
        def body(b, carry):
            o = b * L
            r = [[rows[s][p][pl.ds(c * C + o, L)]
                  for c in range(3)] for p in range(4)]
            b1 = [r[1][c] - r[0][c] for c in range(3)]
            b2 = [r[2][c] - r[1][c] for c in range(3)]
            b3 = [r[3][c] - r[2][c] for c in range(3)]
            n1 = [b1[1] * b2[2] - b1[2] * b2[1],
                  b1[2] * b2[0] - b1[0] * b2[2],
                  b1[0] * b2[1] - b1[1] * b2[0]]
            n2 = [b2[1] * b3[2] - b2[2] * b3[1],
                  b2[2] * b3[0] - b2[0] * b3[2],
                  b2[0] * b3[1] - b2[1] * b3[0]]
            d = n1[0] * n2[0] + n1[1] * n2[1] + n1[2] * n2[2]
            n1sq = n1[0] * n1[0] + n1[1] * n1[1] + n1[2] * n1[2]
            n2sq = n2[0] * n2[0] + n2[1] * n2[1] + n2[2] * n2[2]
            sdot = n1[0] * b3[0] + n1[1] * b3[1] + n1[2] * b3[2]
            denom = n1sq * n2sq
            sq = denom * _rsqrt(jnp.maximum(denom, 1e-35))
            cos_raw = d / sq
            cos_cl = jnp.minimum(jnp.maximum(cos_raw, -0.999999999), 0.99999999)
            is_nan = cos_raw != cos_raw
            cos = jnp.where(is_nan, cos_raw, cos_cl)
            phi = _acos(cos)
            phi = jnp.where(is_nan, cos, phi)
            phi = jnp.where(sdot > 0, phi, -phi)
            out_v[pl.ds(o, L)] = phi
            return carry

        lax.fori_loop(0, C // L, body, 0)
        pltpu.sync_copy(out_v, out_hbm.at[pl.ds(g * C, C)])

    g0_first = wid

    @pl.when(g0_first < NCHUNKS)
    def _():
        load_and_build(g0_first, 0)
        fire(0)

    def pair(k, carry):
        g0 = wid + (2 * k) * NW
        g1 = g0 + NW
        g2 = g1 + NW

        @pl.when(g1 < NCHUNKS)
        def _():
            load_and_build(g1, 1)

        @pl.when(g0 < NCHUNKS)
        def _():
            drain(0)

        @pl.when(g1 < NCHUNKS)
        def _():
            fire(1)

        @pl.when(g0 < NCHUNKS)
        def _():
            compute(g0, 0)

        @pl.when(g2 < NCHUNKS)
        def _():
            load_and_build(g2, 0)

        @pl.when(g1 < NCHUNKS)
        def _():
            drain(1)

        @pl.when(g2 < NCHUNKS)
        def _():
            fire(0)

        @pl.when(g1 < NCHUNKS)
        def _():
            compute(g1, 1)

        return carry

    lax.fori_loop(0, NPAIRS, pair, 0)


def kernel(coords, torsions):
    coords_t = coords.T.reshape(-1)
    tors_a = torsions[:NT_HALF].T.reshape(-1)
    tors_b = torsions[NT_HALF:].T.reshape(-1)

    mesh = plsc.VectorSubcoreMesh(core_axis_name="c", subcore_axis_name="s")
    run = functools.partial(
        pl.kernel,
        mesh=mesh,
        compiler_params=pltpu.CompilerParams(needs_layout_passes=False,
                                             use_tc_tiling_on_sc=False),
        out_type=jax.ShapeDtypeStruct((NT_HALF,), jnp.float32),
        scratch_types=[
            pltpu.VMEM_SHARED((3 * N_ATOMS,), jnp.float32),
            pltpu.VMEM((4 * C,), jnp.int32),
        ] + [pltpu.VMEM((3 * C,), jnp.int32) for _ in range(8)]
          + [pltpu.VMEM((3 * C,), jnp.float32) for _ in range(8)]
          + [pltpu.VMEM((C,), jnp.float32)]
          + [pltpu.SemaphoreType.DMA for _ in range(8)],
    )(_torsion_sc_kernel)
    out_a = run(tors_a, coords_t)
    out_b = run(tors_b, coords_t)
    return jnp.concatenate([out_a, out_b])


# speedup vs baseline: 1.3347x; 1.1208x over previous
import functools

import jax
import jax.numpy as jnp
from jax import lax
from jax.experimental import pallas as pl
from jax.experimental.pallas import tpu as pltpu
from jax.experimental.pallas import tpu_sc as plsc

N_ATOMS = 500000
N_TORSIONS = 2000000
NT_HALF = N_TORSIONS // 2
L = 16
C = 400
NCHUNKS = NT_HALF // C
NC = 2
NS = 16
NW = NC * NS
ITERS_PER_W = -(-NCHUNKS // NW)
NPAIRS = -(-ITERS_PER_W // 2)
NPIECE = 6000
NPIECES = (3 * N_ATOMS) // NPIECE

_PI = 3.14159265358979


def _rsqrt(y):
    i = plsc.bitcast(y, jnp.int32)
    i = jnp.int32(0x5F3759DF) - (i >> 1)
    r = plsc.bitcast(i, jnp.float32)
    for _ in range(3):
        r = r * (1.5 - 0.5 * y * r * r)
    return r


def _acos(x):
    t = jnp.abs(x)
    y = 1.0 - t
    s = y * _rsqrt(jnp.maximum(y, 1e-30))
    p = -0.0012624911
    for a in (0.0066700901, -0.0170881256, 0.0308918810, -0.0501743046,
              0.0889789874, -0.2145988016, 1.5707963050):
        p = p * t + a
    r = s * p
    return jnp.where(x >= 0, r, _PI - r)


def _torsion_sc_kernel(tors_hbm, coords_hbm, out_hbm,
                       table_sh, idx_v,
                       ga0, ga1, ga2, ga3, gb0, gb1, gb2, gb3,
                       ra0, ra1, ra2, ra3, rb0, rb1, rb2, rb3,
                       stage_v, out_v,
                       sa0, sa1, sa2, sa3, sb0, sb1, sb2, sb3):
    gidx = [[ga0, ga1, ga2, ga3], [gb0, gb1, gb2, gb3]]
    rows = [[ra0, ra1, ra2, ra3], [rb0, rb1, rb2, rb3]]
    sems = [[sa0, sa1, sa2, sa3], [sb0, sb1, sb2, sb3]]
    cid = lax.axis_index("c")
    sid = lax.axis_index("s")
    wid = sid * NC + cid

    def stage(q0, carry):
        q = sid + q0 * NS

        @pl.when(q < NPIECES)
        def _():
            pltpu.sync_copy(coords_hbm.at[pl.ds(q * NPIECE, NPIECE)], stage_v)
            pltpu.sync_copy(stage_v, table_sh.at[pl.ds(q * NPIECE, NPIECE)])

        return carry

    lax.fori_loop(0, -(-NPIECES // NS), stage, 0)
    plsc.subcore_barrier()

    def load_and_build(g, s):
        for p in range(4):
            pltpu.sync_copy(tors_hbm.at[pl.ds(p * NT_HALF + g * C, C)],
                            idx_v.at[pl.ds(p * C, C)])

        def build(b, carry):
            o = b * L
            for p in range(4):
                ap = idx_v[pl.ds(p * C + o, L)]
                for c in range(3):
                    gidx[s][p][pl.ds(c * C + o, L)] = ap + c * N_ATOMS
            return carry

        lax.fori_loop(0, C // L, build, 0)

    def fire(s):
        for p in range(4):
            pltpu.async_copy(table_sh.at[gidx[s][p]], rows[s][p], sems[s][p])

    def drain(s):
        for p in range(4):
            pltpu.make_async_copy(table_sh.at[gidx[s][p]], rows[s][p],
                                  sems[s][p]).wait()

    def compute(g, s):
        def body(b, carry):
            o = b * L
            r = [[rows[s][p][pl.ds(c * C + o, L)]
                  for c in range(3)] for p in range(4)]
            b1 = [r[1][c] - r[0][c] for c in range(3)]
            b2 = [r[2][c] - r[1][c] for c in range(3)]
            b3 = [r[3][c] - r[2][c] for c in range(3)]
            n1 = [b1[1] * b2[2] - b1[2] * b2[1],
                  b1[2] * b2[0] - b1[0] * b2[2],
                  b1[0] * b2[1] - b1[1] * b2[0]]
            n2 = [b2[1] * b3[2] - b2[2] * b3[1],
                  b2[2] * b3[0] - b2[0] * b3[2],
                  b2[0] * b3[1] - b2[1] * b3[0]]
            d = n1[0] * n2[0] + n1[1] * n2[1] + n1[2] * n2[2]
            n1sq = n1[0] * n1[0] + n1[1] * n1[1] + n1[2] * n1[2]
            n2sq = n2[0] * n2[0] + n2[1] * n2[1] + n2[2] * n2[2]
            sdot = n1[0] * b3[0] + n1[1] * b3[1] + n1[2] * b3[2]
            denom = n1sq * n2sq
            sq = denom * _rsqrt(jnp.maximum(denom, 1e-35))
            cos_raw = d / sq
            cos_cl = jnp.minimum(jnp.maximum(cos_raw, -0.999999999), 0.99999999)
            is_nan = cos_raw != cos_raw
            cos = jnp.where(is_nan, cos_raw, cos_cl)
            phi = _acos(cos)
            phi = jnp.where(is_nan, cos, phi)
            phi = jnp.where(sdot > 0, phi, -phi)
            out_v[pl.ds(o, L)] = phi
            return carry

        lax.fori_loop(0, C // L, body, 0)
        pltpu.sync_copy(out_v, out_hbm.at[pl.ds(g * C, C)])

    g0_first = wid

    @pl.when(g0_first < NCHUNKS)
    def _():
        load_and_build(g0_first, 0)
        fire(0)

    def pair(k, carry):
        g0 = wid + (2 * k) * NW
        g1 = g0 + NW
        g2 = g1 + NW

        @pl.when(g1 < NCHUNKS)
        def _():
            load_and_build(g1, 1)

        @pl.when(g0 < NCHUNKS)
        def _():
            drain(0)

        @pl.when(g1 < NCHUNKS)
        def _():
            fire(1)

        @pl.when(g0 < NCHUNKS)
        def _():
            compute(g0, 0)

        @pl.when(g2 < NCHUNKS)
        def _():
            load_and_build(g2, 0)

        @pl.when(g1 < NCHUNKS)
        def _():
            drain(1)

        @pl.when(g2 < NCHUNKS)
        def _():
            fire(0)

        @pl.when(g1 < NCHUNKS)
        def _():
            compute(g1, 1)

        return carry

    lax.fori_loop(0, NPAIRS, pair, 0)


def kernel(coords, torsions):
    coords_t = coords.T.reshape(-1)
    tors_a = torsions[:NT_HALF].T.reshape(-1)
    tors_b = torsions[NT_HALF:].T.reshape(-1)

    mesh = plsc.VectorSubcoreMesh(core_axis_name="c", subcore_axis_name="s")
    run = functools.partial(
        pl.kernel,
        mesh=mesh,
        compiler_params=pltpu.CompilerParams(needs_layout_passes=False,
                                             use_tc_tiling_on_sc=False),
        out_type=jax.ShapeDtypeStruct((NT_HALF,), jnp.float32),
        scratch_types=[
            pltpu.VMEM_SHARED((3 * N_ATOMS,), jnp.float32),
            pltpu.VMEM((4 * C,), jnp.int32),
        ] + [pltpu.VMEM((3 * C,), jnp.int32) for _ in range(8)]
          + [pltpu.VMEM((3 * C,), jnp.float32) for _ in range(8)]
          + [pltpu.VMEM((NPIECE,), jnp.float32)]
          + [pltpu.VMEM((C,), jnp.float32)]
          + [pltpu.SemaphoreType.DMA for _ in range(8)],
    )(_torsion_sc_kernel)
    out_a = run(tors_a, coords_t)
    out_b = run(tors_b, coords_t)
    return jnp.concatenate([out_a, out_b])
